# baseline (speedup 1.0000x reference)
import jax
import jax.numpy as jnp
from jax.experimental import pallas as pl
from jax.experimental.pallas import tpu as pltpu

N = 10000
D_IN = 128
D_OUT = 32
V = 2
BM = 200
NBUF = 3


def _outer(seq_ref, w_ref, b_ref, adj_hbm, ha_hbm, hp_hbm, fus_hbm,
           h_scratch, ha_sem):
    h_scratch[...] = (
        jnp.dot(seq_ref[...], w_ref[...].T, preferred_element_type=jnp.float32)
        + b_ref[...]
    )
    ha_copy = pltpu.make_async_copy(h_scratch, ha_hbm, ha_sem)
    ha_copy.start()

    def inner(adj_ref, hp_ref, fus_ref):
        h = h_scratch[...]
        hp0 = jnp.dot(adj_ref[0], h, preferred_element_type=jnp.float32)
        hp1 = jnp.dot(adj_ref[1], h, preferred_element_type=jnp.float32)
        hp_ref[0] = hp0
        hp_ref[1] = hp1
        fus_ref[...] = (hp0 + hp1) * (1.0 / V)

    pipeline = pltpu.emit_pipeline(
        inner,
        grid=(N // BM,),
        in_specs=[
            pl.BlockSpec((V, BM, N), lambda m: (0, m, 0),
                         pipeline_mode=pl.Buffered(buffer_count=NBUF)),
        ],
        out_specs=[
            pl.BlockSpec((V, BM, D_OUT), lambda m: (0, m, 0)),
            pl.BlockSpec((BM, D_OUT), lambda m: (m, 0)),
        ],
    )
    pipeline(adj_hbm, hp_hbm, fus_hbm)
    ha_copy.wait()


@jax.jit
def kernel(seq_a, adj_list, W, b):
    b2 = b.reshape(1, D_OUT)
    h_a, h_p_list, h_p_fusion = pl.pallas_call(
        _outer,
        in_specs=[
            pl.BlockSpec(memory_space=pltpu.MemorySpace.VMEM),
            pl.BlockSpec(memory_space=pltpu.MemorySpace.VMEM),
            pl.BlockSpec(memory_space=pltpu.MemorySpace.VMEM),
            pl.BlockSpec(memory_space=pl.ANY),
        ],
        out_specs=[
            pl.BlockSpec(memory_space=pl.ANY),
            pl.BlockSpec(memory_space=pl.ANY),
            pl.BlockSpec(memory_space=pl.ANY),
        ],
        out_shape=[
            jax.ShapeDtypeStruct((N, D_OUT), jnp.float32),
            jax.ShapeDtypeStruct((V, N, D_OUT), jnp.float32),
            jax.ShapeDtypeStruct((N, D_OUT), jnp.float32),
        ],
        scratch_shapes=[
            pltpu.VMEM((N, D_OUT), jnp.float32),
            pltpu.SemaphoreType.DMA,
        ],
    )(seq_a, W, b2, adj_list)

    return (h_a, h_p_list, h_p_fusion)


# speedup vs baseline: 1.0179x; 1.0179x over previous
import jax
import jax.numpy as jnp
from jax.experimental import pallas as pl
from jax.experimental.pallas import tpu as pltpu

N = 10000
D_IN = 128
D_OUT = 32
V = 2
BM = 200


def _mlp_kernel(seq_ref, w_ref, b_ref, out_ref):
    out_ref[...] = (
        jnp.dot(seq_ref[...], w_ref[...].T, preferred_element_type=jnp.float32)
        + b_ref[...]
    )


def _agg_kernel(adj_ref, h_ref, hp_ref, fus_ref):
    h = h_ref[...]
    hp0 = jnp.dot(adj_ref[0], h, preferred_element_type=jnp.float32)
    hp1 = jnp.dot(adj_ref[1], h, preferred_element_type=jnp.float32)
    hp_ref[0] = hp0
    hp_ref[1] = hp1
    fus_ref[...] = (hp0 + hp1) * (1.0 / V)


@jax.jit
def kernel(seq_a, adj_list, W, b):
    b2 = b.reshape(1, D_OUT)
    h_a = pl.pallas_call(
        _mlp_kernel,
        out_shape=jax.ShapeDtypeStruct((N, D_OUT), jnp.float32),
    )(seq_a, W, b2)

    h_p_list, h_p_fusion = pl.pallas_call(
        _agg_kernel,
        grid=(N // BM,),
        in_specs=[
            pl.BlockSpec((V, BM, N), lambda m: (0, m, 0)),
            pl.BlockSpec((N, D_OUT), lambda m: (0, 0)),
        ],
        out_specs=[
            pl.BlockSpec((V, BM, D_OUT), lambda m: (0, m, 0)),
            pl.BlockSpec((BM, D_OUT), lambda m: (m, 0)),
        ],
        out_shape=[
            jax.ShapeDtypeStruct((V, N, D_OUT), jnp.float32),
            jax.ShapeDtypeStruct((N, D_OUT), jnp.float32),
        ],
        compiler_params=pltpu.CompilerParams(
            dimension_semantics=("parallel",),
        ),
    )(adj_list, h_a)

    return (h_a, h_p_list, h_p_fusion)
